# Initial kernel scaffold; baseline (speedup 1.0000x reference)
import functools

import jax
import jax.numpy as jnp
from jax import lax
from jax.experimental import pallas as pl
from jax.experimental.pallas import tpu as pltpu
from jax.experimental.pallas import tpu_sc as plsc

N = 10000
E = 320000
D = 128

NC = 2
NS = 16
NW = NC * NS
EPW = E // NW
CHUNK = 400
NCHUNK = EPW // CHUNK
RPT = N // NS


def _proj_body(x_ref, wt_ref, a8_ref, wh_ref, s12_ref):
    wh = jnp.dot(x_ref[...], wt_ref[...], preferred_element_type=jnp.float32)
    wh_ref[...] = wh
    s12_ref[...] = jnp.dot(wh, a8_ref[...], preferred_element_type=jnp.float32)


def _proj(x, wt, a8):
    B = 1000
    return pl.pallas_call(
        _proj_body,
        grid=(N // B,),
        in_specs=[
            pl.BlockSpec((B, D), lambda i: (i, 0)),
            pl.BlockSpec((D, D), lambda i: (0, 0)),
            pl.BlockSpec((D, D), lambda i: (0, 0)),
        ],
        out_specs=[
            pl.BlockSpec((B, D), lambda i: (i, 0)),
            pl.BlockSpec((B, D), lambda i: (i, 0)),
        ],
        out_shape=[
            jax.ShapeDtypeStruct((N, D), jnp.float32),
            jax.ShapeDtypeStruct((N, D), jnp.float32),
        ],
    )(x, wt, a8)


def _edge_body(src_hbm, dst_hbm, wh_hbm, s1_hbm, s2_hbm,
               out_hbm, alpha_hbm,
               s1_v, s2_v, srcv, dstv, w_v, rows_v, out_acc, alpha_acc, sem):
    c = lax.axis_index("c")
    s = lax.axis_index("s")
    wid = c * NS + s

    pltpu.sync_copy(s1_hbm, s1_v)
    pltpu.sync_copy(s2_hbm, s2_v)

    def _zw(i, carry):
        w_v[pl.ds(i * 16, 16)] = jnp.zeros((16,), jnp.float32)
        return carry

    lax.fori_loop(0, CHUNK // 16, _zw, 0)

    def _zr(i, carry):
        for f in range(D // 16):
            rows_v[i, pl.ds(f * 16, 16)] = jnp.zeros((16,), jnp.float32)
        return carry

    lax.fori_loop(0, CHUNK, _zr, 0)

    pltpu.sync_copy(rows_v, out_acc.at[pl.ds(s * RPT, CHUNK)])
    pltpu.sync_copy(rows_v.at[pl.ds(0, RPT - CHUNK)],
                    out_acc.at[pl.ds(s * RPT + CHUNK, RPT - CHUNK)])

    @pl.when(s == 0)
    def _():
        def _za(k, carry):
            pltpu.sync_copy(w_v, alpha_acc.at[pl.ds(k * CHUNK, CHUNK)])
            return carry
        lax.fori_loop(0, N // CHUNK, _za, 0)

    plsc.subcore_barrier()

    def _chunk(k, carry):
        off = wid * EPW + k * CHUNK
        pltpu.sync_copy(src_hbm.at[pl.ds(off, CHUNK)], srcv)
        pltpu.sync_copy(dst_hbm.at[pl.ds(off, CHUNK)], dstv)
        pltpu.async_copy(wh_hbm.at[srcv], rows_v, sem).wait()

        def _w16(j, carry2):
            sidx = srcv[pl.ds(j * 16, 16)]
            didx = dstv[pl.ds(j * 16, 16)]
            e = plsc.load_gather(s1_v, [sidx]) + plsc.load_gather(s2_v, [didx])
            e = jnp.where(e > 0, e, 0.2 * e)
            w = jnp.exp(e)
            w_v[pl.ds(j * 16, 16)] = w
            for l in range(16):
                wl = w.at[jnp.full((16,), l, jnp.int32)].get(
                    mode="promise_in_bounds")
                r = j * 16 + l
                for f in range(D // 16):
                    rows_v[r, pl.ds(f * 16, 16)] = (
                        rows_v[r, pl.ds(f * 16, 16)] * wl)
            return carry2

        lax.fori_loop(0, CHUNK // 16, _w16, 0)

        pltpu.async_copy(w_v, alpha_acc.at[dstv], sem, add=True).wait()
        pltpu.async_copy(rows_v, out_acc.at[dstv], sem, add=True).wait()
        return carry

    lax.fori_loop(0, NCHUNK, _chunk, 0)

    plsc.subcore_barrier()

    pltpu.sync_copy(out_acc.at[pl.ds(s * RPT, RPT)],
                    out_hbm.at[c, pl.ds(s * RPT, RPT)])

    @pl.when(s == 0)
    def _():
        pltpu.sync_copy(alpha_acc, alpha_hbm.at[c])


_edge_kernel = pl.kernel(
    _edge_body,
    out_type=(
        jax.ShapeDtypeStruct((NC, N, D), jnp.float32),
        jax.ShapeDtypeStruct((NC, N), jnp.float32),
    ),
    mesh=plsc.VectorSubcoreMesh(core_axis_name="c", subcore_axis_name="s"),
    scratch_types=[
        pltpu.VMEM((N,), jnp.float32),
        pltpu.VMEM((N,), jnp.float32),
        pltpu.VMEM((CHUNK,), jnp.int32),
        pltpu.VMEM((CHUNK,), jnp.int32),
        pltpu.VMEM((CHUNK,), jnp.float32),
        pltpu.VMEM((CHUNK, D), jnp.float32),
        pltpu.VMEM_SHARED((N, D), jnp.float32),
        pltpu.VMEM_SHARED((N,), jnp.float32),
        pltpu.SemaphoreType.DMA,
    ],
)


def _final_body(p_ref, at_ref, o_ref):
    ps = p_ref[0] + p_ref[1]
    a = at_ref[:, 0] + at_ref[:, 1]
    a = jnp.maximum(a, 1e-10)
    o_ref[...] = ps / a[:, None]


def _final(out_parts, alpha_t):
    B = 1000
    return pl.pallas_call(
        _final_body,
        grid=(N // B,),
        in_specs=[
            pl.BlockSpec((NC, B, D), lambda i: (0, i, 0)),
            pl.BlockSpec((B, NC), lambda i: (i, 0)),
        ],
        out_specs=pl.BlockSpec((B, D), lambda i: (i, 0)),
        out_shape=jax.ShapeDtypeStruct((N, D), jnp.float32),
    )(out_parts, alpha_t)


def kernel(x, edge_index, W, attn_w):
    src = edge_index[0].astype(jnp.int32)
    dst = edge_index[1].astype(jnp.int32)
    wt = W.T
    a1 = attn_w[0, :D]
    a2 = attn_w[0, D:]
    a8 = jnp.zeros((D, D), jnp.float32).at[:, 0].set(a1).at[:, 1].set(a2)
    wh, s12 = _proj(x, wt, a8)
    s1 = s12[:, 0]
    s2 = s12[:, 1]
    out_parts, alpha_parts = _edge_kernel(src, dst, wh, s1, s2)
    return _final(out_parts, alpha_parts.T)


# speedup vs baseline: 9.4868x; 9.4868x over previous
import jax
import jax.numpy as jnp
from jax import lax
from jax.experimental import pallas as pl
from jax.experimental.pallas import tpu as pltpu
from jax.experimental.pallas import tpu_sc as plsc

N = 10000
E = 320000
D = 128

NC = 2
NS = 16
NPC = N // NC
ACC_ROWS = 5008
EPW = E // NS
CHUNK = 400
NCHUNK = EPW // CHUNK
RPT = 312


def _proj_body(x_ref, wt_ref, a8_ref, wh_ref, s12_ref):
    wh = jnp.dot(x_ref[...], wt_ref[...], preferred_element_type=jnp.float32)
    wh_ref[...] = wh
    s12_ref[...] = jnp.dot(wh, a8_ref[...], preferred_element_type=jnp.float32)


def _proj(x, wt, a8):
    B = 1000
    return pl.pallas_call(
        _proj_body,
        grid=(N // B,),
        in_specs=[
            pl.BlockSpec((B, D), lambda i: (i, 0)),
            pl.BlockSpec((D, D), lambda i: (0, 0)),
            pl.BlockSpec((D, D), lambda i: (0, 0)),
        ],
        out_specs=[
            pl.BlockSpec((B, D), lambda i: (i, 0)),
            pl.BlockSpec((B, D), lambda i: (i, 0)),
        ],
        out_shape=[
            jax.ShapeDtypeStruct((N, D), jnp.float32),
            jax.ShapeDtypeStruct((N, D), jnp.float32),
        ],
    )(x, wt, a8)


def _edge_body(src_hbm, dst_hbm, wh_hbm, s1_hbm, s2_hbm,
               out_hbm, alpha_hbm,
               s1_v, s2_v, srcv, dstv, dstv2, w_v, wm_v, rows_v,
               out_acc, alpha_acc, sem):
    c = lax.axis_index("c")
    s = lax.axis_index("s")
    base = c * NPC

    pltpu.sync_copy(s1_hbm, s1_v)
    pltpu.sync_copy(s2_hbm, s2_v)

    def _zw(i, carry):
        w_v[pl.ds(i * 16, 16)] = jnp.zeros((16,), jnp.float32)
        return carry

    lax.fori_loop(0, CHUNK // 16, _zw, 0)

    def _zr(i, carry):
        for f in range(D // 16):
            rows_v[i, pl.ds(f * 16, 16)] = jnp.zeros((16,), jnp.float32)
        return carry

    lax.fori_loop(0, CHUNK, _zr, 0)

    pltpu.sync_copy(rows_v.at[pl.ds(0, RPT)], out_acc.at[pl.ds(s * RPT, RPT)])

    @pl.when(s == NS - 1)
    def _():
        rem = ACC_ROWS - NS * RPT
        pltpu.sync_copy(rows_v.at[pl.ds(0, rem)],
                        out_acc.at[pl.ds(NS * RPT, rem)])

    @pl.when(jnp.logical_and(c == 0, s == 0))
    def _():
        def _za(k, carry):
            pltpu.sync_copy(w_v, alpha_acc.at[pl.ds(k * CHUNK, CHUNK)])
            return carry
        lax.fori_loop(0, N // CHUNK, _za, 0)

    plsc.subcore_barrier()

    def _chunk(k, carry):
        off = s * EPW + k * CHUNK
        pltpu.sync_copy(src_hbm.at[pl.ds(off, CHUNK)], srcv)
        pltpu.sync_copy(dst_hbm.at[pl.ds(off, CHUNK)], dstv)

        pltpu.async_copy(wh_hbm.at[srcv], rows_v, sem).wait()

        def _w16(j, carry2):
            sidx = srcv[pl.ds(j * 16, 16)]
            didx = dstv[pl.ds(j * 16, 16)]
            e = plsc.load_gather(s1_v, [sidx]) + plsc.load_gather(s2_v, [didx])
            e = jnp.where(e > 0, e, 0.2 * e)
            w = jnp.exp(e)
            lo = didx - base
            valid = jnp.logical_and(lo >= 0, lo < NPC)
            w_v[pl.ds(j * 16, 16)] = w
            wm_v[pl.ds(j * 16, 16)] = jnp.where(valid, w, 0.0)
            dstv2[pl.ds(j * 16, 16)] = jnp.where(valid, lo, 0)
            return carry2

        lax.fori_loop(0, CHUNK // 16, _w16, 0)

        def _sc16(j, carry2):
            w = wm_v[pl.ds(j * 16, 16)]
            for l in range(16):
                wl = w.at[jnp.full((16,), l, jnp.int32)].get(
                    mode="promise_in_bounds")
                r = j * 16 + l
                for f in range(D // 16):
                    rows_v[r, pl.ds(f * 16, 16)] = (
                        rows_v[r, pl.ds(f * 16, 16)] * wl)
            return carry2

        lax.fori_loop(0, CHUNK // 16, _sc16, 0)

        pltpu.async_copy(rows_v, out_acc.at[dstv2], sem, add=True).wait()

        @pl.when(c == 0)
        def _():
            pltpu.async_copy(w_v, alpha_acc.at[dstv], sem, add=True).wait()

        return carry

    lax.fori_loop(0, NCHUNK, _chunk, 0)

    plsc.subcore_barrier()

    pltpu.sync_copy(out_acc.at[pl.ds(s * RPT, RPT)],
                    out_hbm.at[c, pl.ds(s * RPT, RPT)])

    @pl.when(s == NS - 1)
    def _():
        rem = ACC_ROWS - NS * RPT
        pltpu.sync_copy(out_acc.at[pl.ds(NS * RPT, rem)],
                        out_hbm.at[c, pl.ds(NS * RPT, rem)])

    @pl.when(jnp.logical_and(c == 0, s == 0))
    def _():
        pltpu.sync_copy(alpha_acc, alpha_hbm.at[0])


_edge_kernel = pl.kernel(
    _edge_body,
    out_type=(
        jax.ShapeDtypeStruct((NC, ACC_ROWS, D), jnp.float32),
        jax.ShapeDtypeStruct((8, N), jnp.float32),
    ),
    mesh=plsc.VectorSubcoreMesh(core_axis_name="c", subcore_axis_name="s"),
    compiler_params=pltpu.CompilerParams(needs_layout_passes=False),
    scratch_types=[
        pltpu.VMEM((N,), jnp.float32),
        pltpu.VMEM((N,), jnp.float32),
        pltpu.VMEM((CHUNK,), jnp.int32),
        pltpu.VMEM((CHUNK,), jnp.int32),
        pltpu.VMEM((CHUNK,), jnp.int32),
        pltpu.VMEM((CHUNK,), jnp.float32),
        pltpu.VMEM((CHUNK,), jnp.float32),
        pltpu.VMEM((CHUNK, D), jnp.float32),
        pltpu.VMEM_SHARED((ACC_ROWS, D), jnp.float32),
        pltpu.VMEM_SHARED((N,), jnp.float32),
        pltpu.SemaphoreType.DMA,
    ],
)


def _final_body(p_ref, at_ref, o_ref):
    a = jnp.maximum(at_ref[:, 0], 1e-10)
    o_ref[...] = p_ref[...] / a[:, None]


def _final(out_cat, alpha_t):
    B = 1000
    return pl.pallas_call(
        _final_body,
        grid=(N // B,),
        in_specs=[
            pl.BlockSpec((B, D), lambda i: (i, 0)),
            pl.BlockSpec((B, 8), lambda i: (i, 0)),
        ],
        out_specs=pl.BlockSpec((B, D), lambda i: (i, 0)),
        out_shape=jax.ShapeDtypeStruct((N, D), jnp.float32),
    )(out_cat, alpha_t)


def kernel(x, edge_index, W, attn_w):
    src = edge_index[0].astype(jnp.int32)
    dst = edge_index[1].astype(jnp.int32)
    wt = W.T
    a1 = attn_w[0, :D]
    a2 = attn_w[0, D:]
    a8 = jnp.zeros((D, D), jnp.float32).at[:, 0].set(a1).at[:, 1].set(a2)
    wh, s12 = _proj(x, wt, a8)
    s1 = s12[:, 0]
    s2 = s12[:, 1]
    out_parts, alpha_parts = _edge_kernel(src, dst, wh, s1, s2)
    out_cat = jnp.concatenate(
        [out_parts[0, :NPC], out_parts[1, :NPC]], axis=0)
    return _final(out_cat, alpha_parts.T)
